# baseline (speedup 1.0000x reference)
import jax
import jax.numpy as jnp
from jax.experimental import pallas as pl
from jax.experimental.pallas import tpu as pltpu

_TM = 400


def _gcn_body(feat_ref, w_ref, adj_a_ref, adj_b_ref, bias_ref, out_ref,
              fw_ref):
    step = pl.program_id(0)
    h = adj_a_ref.shape[0]

    @pl.when(step == 0)
    def _():
        fw_ref[...] = jnp.dot(feat_ref[...], w_ref[...],
                              preferred_element_type=jnp.float32)

    fw = fw_ref[...]
    acc_a = jnp.dot(adj_a_ref[...], fw, preferred_element_type=jnp.float32)
    out_ref[:h, :] = jnp.maximum(acc_a + bias_ref[:h, :], 0.0)
    acc_b = jnp.dot(adj_b_ref[...], fw, preferred_element_type=jnp.float32)
    out_ref[h:, :] = jnp.maximum(acc_b + bias_ref[h:, :], 0.0)


def kernel(adj, feature, weight, bias):
    n, d = feature.shape
    f = weight.shape[1]
    tm = _TM
    th = tm // 2
    grid = (n // tm,)
    return pl.pallas_call(
        _gcn_body,
        grid=grid,
        in_specs=[
            pl.BlockSpec((n, d), lambda i: (0, 0)),
            pl.BlockSpec((d, f), lambda i: (0, 0)),
            pl.BlockSpec((th, n), lambda i: (2 * i, 0)),
            pl.BlockSpec((th, n), lambda i: (2 * i + 1, 0)),
            pl.BlockSpec((tm, f), lambda i: (i, 0)),
        ],
        out_specs=pl.BlockSpec((tm, f), lambda i: (i, 0)),
        out_shape=jax.ShapeDtypeStruct((n, f), jnp.float32),
        scratch_shapes=[pltpu.VMEM((n, f), jnp.float32)],
    )(feature, weight, adj, adj, bias)


# speedup vs baseline: 1.0276x; 1.0276x over previous
import jax
import jax.numpy as jnp
from jax.experimental import pallas as pl
from jax.experimental.pallas import tpu as pltpu

_TM = 200
_NBUF = 3


def _gcn_body(feat_hbm, w_hbm, adj_hbm, bias_hbm, out_hbm,
              bufs, featv, wv, biasv, fwv, outv, adj_sems, aux_sem, out_sem):
    n = adj_hbm.shape[0]
    tm = bufs.shape[1]
    nbuf = bufs.shape[0]
    nblk = n // tm

    for b in range(min(nbuf, nblk)):
        pltpu.make_async_copy(adj_hbm.at[pl.ds(b * tm, tm), :],
                              bufs.at[b], adj_sems.at[b]).start()
    cp_feat = pltpu.make_async_copy(feat_hbm, featv, aux_sem)
    cp_feat.start()
    cp_w = pltpu.make_async_copy(w_hbm, wv, aux_sem)
    cp_w.start()
    cp_bias = pltpu.make_async_copy(bias_hbm, biasv, aux_sem)
    cp_bias.start()
    cp_feat.wait()
    cp_w.wait()
    cp_bias.wait()

    fwv[...] = jnp.dot(featv[...], wv[...], preferred_element_type=jnp.float32)

    def step(i, carry):
        slot = jax.lax.rem(i, nbuf)
        pltpu.make_async_copy(adj_hbm.at[pl.ds(i * tm, tm), :],
                              bufs.at[slot], adj_sems.at[slot]).wait()
        acc = jnp.dot(bufs[slot], fwv[...], preferred_element_type=jnp.float32)
        outv[pl.ds(i * tm, tm), :] = jnp.maximum(
            acc + biasv[pl.ds(i * tm, tm), :], 0.0)

        @pl.when(i + nbuf < nblk)
        def _():
            pltpu.make_async_copy(adj_hbm.at[pl.ds((i + nbuf) * tm, tm), :],
                                  bufs.at[slot], adj_sems.at[slot]).start()
        return carry

    jax.lax.fori_loop(0, nblk, step, 0)

    cp_out = pltpu.make_async_copy(outv, out_hbm, out_sem)
    cp_out.start()
    cp_out.wait()


def kernel(adj, feature, weight, bias):
    n, d = feature.shape
    f = weight.shape[1]
    tm, nbuf = _TM, _NBUF
    any_spec = pl.BlockSpec(memory_space=pltpu.MemorySpace.HBM)
    return pl.pallas_call(
        _gcn_body,
        in_specs=[any_spec, any_spec, any_spec, any_spec],
        out_specs=pl.BlockSpec(memory_space=pltpu.MemorySpace.HBM),
        out_shape=jax.ShapeDtypeStruct((n, f), jnp.float32),
        scratch_shapes=[
            pltpu.VMEM((nbuf, tm, n), jnp.float32),
            pltpu.VMEM((n, d), jnp.float32),
            pltpu.VMEM((d, f), jnp.float32),
            pltpu.VMEM((n, f), jnp.float32),
            pltpu.VMEM((n, f), jnp.float32),
            pltpu.VMEM((n, f), jnp.float32),
            pltpu.SemaphoreType.DMA((nbuf,)),
            pltpu.SemaphoreType.DMA,
            pltpu.SemaphoreType.DMA,
        ],
    )(feature, weight, adj, bias)


# speedup vs baseline: 1.0350x; 1.0073x over previous
import jax
import jax.numpy as jnp
from jax.experimental import pallas as pl
from jax.experimental.pallas import tpu as pltpu

_TM = 200
_NBUF = 3


def _gcn_body(feat_hbm, w_hbm, adj_hbm, bias_hbm, out_hbm,
              bufs, featv, wv, biasv, fwv, outv,
              adj_sems, aux_sem, out_sems):
    n = adj_hbm.shape[0]
    tm = bufs.shape[1]
    nbuf = bufs.shape[0]
    nblk = n // tm

    cp_feat = pltpu.make_async_copy(feat_hbm, featv, aux_sem)
    cp_feat.start()
    cp_w = pltpu.make_async_copy(w_hbm, wv, aux_sem)
    cp_w.start()
    pltpu.make_async_copy(adj_hbm.at[pl.ds(0, tm), :],
                          bufs.at[0], adj_sems.at[0]).start()
    cp_bias = pltpu.make_async_copy(bias_hbm, biasv, aux_sem)
    cp_bias.start()
    for b in range(1, min(nbuf, nblk)):
        pltpu.make_async_copy(adj_hbm.at[pl.ds(b * tm, tm), :],
                              bufs.at[b], adj_sems.at[b]).start()
    cp_feat.wait()
    cp_w.wait()

    fwv[...] = jnp.dot(featv[...], wv[...], preferred_element_type=jnp.float32)
    cp_bias.wait()

    def step(i, carry):
        slot = jax.lax.rem(i, nbuf)
        pltpu.make_async_copy(adj_hbm.at[pl.ds(i * tm, tm), :],
                              bufs.at[slot], adj_sems.at[slot]).wait()
        acc = jnp.dot(bufs[slot], fwv[...], preferred_element_type=jnp.float32)

        @pl.when(i >= nbuf)
        def _():
            pltpu.make_async_copy(outv.at[slot],
                                  out_hbm.at[pl.ds((i - nbuf) * tm, tm), :],
                                  out_sems.at[slot]).wait()

        outv[slot] = jnp.maximum(acc + biasv[pl.ds(i * tm, tm), :], 0.0)
        pltpu.make_async_copy(outv.at[slot],
                              out_hbm.at[pl.ds(i * tm, tm), :],
                              out_sems.at[slot]).start()

        @pl.when(i + nbuf < nblk)
        def _():
            pltpu.make_async_copy(adj_hbm.at[pl.ds((i + nbuf) * tm, tm), :],
                                  bufs.at[slot], adj_sems.at[slot]).start()
        return carry

    jax.lax.fori_loop(0, nblk, step, 0)

    for b in range(min(nbuf, nblk)):
        i = nblk - min(nbuf, nblk) + b
        slot = i % nbuf
        pltpu.make_async_copy(outv.at[slot],
                              out_hbm.at[pl.ds(i * tm, tm), :],
                              out_sems.at[slot]).wait()


def kernel(adj, feature, weight, bias):
    n, d = feature.shape
    f = weight.shape[1]
    tm, nbuf = _TM, _NBUF
    hbm = pl.BlockSpec(memory_space=pltpu.MemorySpace.HBM)
    return pl.pallas_call(
        _gcn_body,
        in_specs=[hbm, hbm, hbm, hbm],
        out_specs=pl.BlockSpec(memory_space=pltpu.MemorySpace.HBM),
        out_shape=jax.ShapeDtypeStruct((n, f), jnp.float32),
        scratch_shapes=[
            pltpu.VMEM((nbuf, tm, n), jnp.float32),
            pltpu.VMEM((n, d), jnp.float32),
            pltpu.VMEM((d, f), jnp.float32),
            pltpu.VMEM((n, f), jnp.float32),
            pltpu.VMEM((n, f), jnp.float32),
            pltpu.VMEM((nbuf, tm, f), jnp.float32),
            pltpu.SemaphoreType.DMA((nbuf,)),
            pltpu.SemaphoreType.DMA,
            pltpu.SemaphoreType.DMA((nbuf,)),
        ],
    )(feature, weight, adj, bias)
